# speedup vs baseline: 1.6290x; 1.6290x over previous
import jax
import jax.numpy as jnp
from jax.experimental import pallas as pl


def _gat(x, src, dst, W, a_s, a_d, b, n):
    h = x @ W
    alpha_src = h @ a_s
    alpha_dst = h @ a_d
    e = alpha_src[src] + alpha_dst[dst]
    e = jnp.where(e > 0, e, 0.2 * e)
    ee = jnp.exp(e)
    denom = jax.ops.segment_sum(ee, dst, num_segments=n)
    raw = jax.ops.segment_sum(ee[:, None] * h[src], dst, num_segments=n)
    return raw / (denom[:, None] + 1e-16) + b


def _fc_body(h_ref, w_ref, b_ref, o_ref):
    o_ref[...] = h_ref[...] @ w_ref[...] + b_ref[...]


def kernel(x, edge_index, W1, a1s, a1d, b1, W2, a2s, a2d, b2, W3, a3s, a3d, b3, Wfc, bfc):
    ei = edge_index.T
    src, dst = ei[0], ei[1]
    n = x.shape[0]
    h = jax.nn.relu(_gat(x, src, dst, W1, a1s, a1d, b1, n))
    h = jax.nn.relu(_gat(h, src, dst, W2, a2s, a2d, b2, n))
    h = jax.nn.relu(_gat(h, src, dst, W3, a3s, a3d, b3, n))
    out = pl.pallas_call(
        _fc_body,
        out_shape=jax.ShapeDtypeStruct((n, 1), jnp.float32),
    )(h, Wfc, bfc[None, :])
    return out


# speedup vs baseline: 51.3448x; 31.5190x over previous
import functools

import jax
import jax.numpy as jnp
from jax import lax
from jax.experimental import pallas as pl
from jax.experimental.pallas import tpu as pltpu
from jax.experimental.pallas import tpu_sc as plsc

_N = 10000
_E = 320000


def _exp_f32(x):
    t = x * 1.4426950408889634
    nf = (t + 12582912.0) - 12582912.0
    f = (t - nf) * 0.6931471805599453
    p = 1.0 + f * (1.0 + f * (0.5 + f * (1.0 / 6.0 + f * (
        1.0 / 24.0 + f * (1.0 / 120.0 + f * (1.0 / 720.0))))))
    n = nf.astype(jnp.int32)
    scale = jax.lax.bitcast_convert_type(
        jax.lax.shift_left(n + 127, 23), jnp.float32)
    return p * scale
_NC = 2
_NS = 16
_NW = _NC * _NS
_C = 128
_NCH = 79
_EPT = _NCH * _C
_EPAD = _NW * _EPT
_NP = 10240
_RPT = _NP // _NS
_F = 16


def _sc_edge_pass(src3, dst3, hx, as_p, ad_p):
    mesh = plsc.VectorSubcoreMesh(core_axis_name="c", subcore_axis_name="s")

    @functools.partial(
        pl.kernel,
        mesh=mesh,
        compiler_params=pltpu.CompilerParams(needs_layout_passes=False,
                                             use_tc_tiling_on_sc=False),
        out_type=[
            jax.ShapeDtypeStruct((_NC, _NP, _F), jnp.float32),
            jax.ShapeDtypeStruct((_NC, _NP), jnp.float32),
        ],
        scratch_types=[
            pltpu.VMEM((_NCH, _C), jnp.int32),
            pltpu.VMEM((_NCH, _C), jnp.int32),
            pltpu.VMEM((_NP,), jnp.float32),
            pltpu.VMEM((_NP,), jnp.float32),
            pltpu.VMEM((_C, _F), jnp.float32),
            pltpu.VMEM((_C,), jnp.float32),
            pltpu.VMEM_SHARED((_NP, _F), jnp.float32),
            pltpu.VMEM_SHARED((_NP,), jnp.float32),
            pltpu.SemaphoreType.DMA,
        ],
    )
    def kern(src_h, dst_h, hx_h, as_h, ad_h, raw_h, den_h,
             src_t, dst_t, as_t, ad_t, rows_t, ee_t, raw_s, den_s, sem):
        cid = lax.axis_index("c")
        sid = lax.axis_index("s")
        wid = sid * _NC + cid

        pltpu.sync_copy(src_h.at[wid], src_t)
        pltpu.sync_copy(dst_h.at[wid], dst_t)
        pltpu.sync_copy(as_h, as_t)
        pltpu.sync_copy(ad_h, ad_t)

        zf = jnp.zeros((_L16,), jnp.float32)

        def zrow(r, _):
            rows_t[r] = zf
            return 0
        lax.fori_loop(0, _C, zrow, 0)

        def zee(k, _):
            ee_t[pl.ds(k * 16, 16)] = zf
            return 0
        lax.fori_loop(0, _C // 16, zee, 0)

        base = sid * _RPT
        for t in range(_RPT // _C):
            pltpu.sync_copy(rows_t, raw_s.at[pl.ds(base + t * _C, _C)])
            pltpu.sync_copy(ee_t, den_s.at[pl.ds(base + t * _C, _C)])
        plsc.subcore_barrier()

        def chunk_body(j, _):
            cp = pltpu.async_copy(hx_h.at[src_t.at[j]], rows_t, sem)

            def ee_body(k, _):
                sidx = src_t[j, pl.ds(k * 16, 16)]
                didx = dst_t[j, pl.ds(k * 16, 16)]
                av = plsc.load_gather(as_t, [sidx])
                dv = plsc.load_gather(ad_t, [didx])
                e = av + dv
                e = jnp.where(e > 0.0, e, 0.2 * e)
                ee_t[pl.ds(k * 16, 16)] = _exp_f32(e)
                return 0
            lax.fori_loop(0, _C // 16, ee_body, 0)
            cp.wait()

            def scale_body(r, _):
                eev = plsc.load_gather(ee_t, [jnp.full((16,), r, jnp.int32)])
                rows_t[r] = rows_t[r] * eev
                return 0
            lax.fori_loop(0, _C, scale_body, 0)

            pltpu.sync_copy(rows_t, raw_s.at[dst_t.at[j]], add=True)
            pltpu.sync_copy(ee_t, den_s.at[dst_t.at[j]], add=True)
            return 0

        lax.fori_loop(0, _NCH, chunk_body, 0)
        plsc.subcore_barrier()

        pltpu.sync_copy(raw_s.at[pl.ds(base, _RPT)],
                        raw_h.at[cid, pl.ds(base, _RPT)])
        pltpu.sync_copy(den_s.at[pl.ds(base, _RPT)],
                        den_h.at[cid, pl.ds(base, _RPT)])

    return kern(src3, dst3, hx, as_p, ad_p)


_L16 = 16


def _dense_first(x, W1, a1s, a1d):
    def body(x_ref, w_ref, as_ref, ad_ref, hx_ref, asp_ref, adp_ref):
        h = jnp.dot(x_ref[...], w_ref[...], preferred_element_type=jnp.float32)
        hx_ref[...] = jnp.zeros((_NP, _F), jnp.float32)
        hx_ref[:_N, :] = h
        asp_ref[...] = jnp.zeros((_NP,), jnp.float32)
        adp_ref[...] = jnp.zeros((_NP,), jnp.float32)
        asp_ref[:_N] = h @ as_ref[...]
        adp_ref[:_N] = h @ ad_ref[...]

    return pl.pallas_call(
        body,
        out_shape=[
            jax.ShapeDtypeStruct((_NP, _F), jnp.float32),
            jax.ShapeDtypeStruct((_NP,), jnp.float32),
            jax.ShapeDtypeStruct((_NP,), jnp.float32),
        ],
    )(x, W1, a1s, a1d)


def _dense_mid(raw, den, b, W, a_s, a_d, fin):
    fout = W.shape[1]

    def body(raw_ref, den_ref, b_ref, w_ref, as_ref, ad_ref,
             hx_ref, asp_ref, adp_ref):
        rawv = raw_ref[0, :_N, :fin] + raw_ref[1, :_N, :fin]
        denv = den_ref[0, :_N] + den_ref[1, :_N]
        z = rawv / (denv[:, None] + 1e-16) + b_ref[...]
        z = jnp.maximum(z, 0.0)
        h = jnp.dot(z, w_ref[...], preferred_element_type=jnp.float32)
        hx_ref[...] = jnp.zeros((_NP, _F), jnp.float32)
        hx_ref[:_N, :fout] = h
        asp_ref[...] = jnp.zeros((_NP,), jnp.float32)
        adp_ref[...] = jnp.zeros((_NP,), jnp.float32)
        asp_ref[:_N] = h @ as_ref[...]
        adp_ref[:_N] = h @ ad_ref[...]

    return pl.pallas_call(
        body,
        out_shape=[
            jax.ShapeDtypeStruct((_NP, _F), jnp.float32),
            jax.ShapeDtypeStruct((_NP,), jnp.float32),
            jax.ShapeDtypeStruct((_NP,), jnp.float32),
        ],
    )(raw, den, b, W, a_s, a_d)


def _dense_last(raw, den, b, Wfc, bfc):
    def body(raw_ref, den_ref, b_ref, w_ref, bfc_ref, o_ref):
        rawv = raw_ref[0, :_N, :8] + raw_ref[1, :_N, :8]
        denv = den_ref[0, :_N] + den_ref[1, :_N]
        z = rawv / (denv[:, None] + 1e-16) + b_ref[...]
        z = jnp.maximum(z, 0.0)
        o_ref[...] = jnp.dot(z, w_ref[...],
                             preferred_element_type=jnp.float32) + bfc_ref[...]

    return pl.pallas_call(
        body,
        out_shape=jax.ShapeDtypeStruct((_N, 1), jnp.float32),
    )(raw, den, b, Wfc, bfc[None, :])


def kernel(x, edge_index, W1, a1s, a1d, b1, W2, a2s, a2d, b2, W3, a3s, a3d, b3, Wfc, bfc):
    src = edge_index[:, 0]
    dst = edge_index[:, 1]
    padn = jnp.full((_EPAD - _E,), _N, jnp.int32)
    src3 = jnp.concatenate([src, padn]).reshape(_NW, _NCH, _C)
    dst3 = jnp.concatenate([dst, padn]).reshape(_NW, _NCH, _C)

    hx, asp, adp = _dense_first(x, W1, a1s, a1d)
    raw, den = _sc_edge_pass(src3, dst3, hx, asp, adp)
    hx, asp, adp = _dense_mid(raw, den, b1, W2, a2s, a2d, _F)
    raw, den = _sc_edge_pass(src3, dst3, hx, asp, adp)
    hx, asp, adp = _dense_mid(raw, den, b2, W3, a3s, a3d, 8)
    raw, den = _sc_edge_pass(src3, dst3, hx, asp, adp)
    return _dense_last(raw, den, b3, Wfc, bfc)
